# Initial kernel scaffold; baseline (speedup 1.0000x reference)
import jax
import jax.numpy as jnp
from jax.experimental import pallas as pl


def kernel(mask, logits_ulb_s, p_t, label_hist):
    raise NotImplementedError("write your pallas kernel here")


# speedup vs baseline: 1.3348x; 1.3348x over previous
import jax
import jax.numpy as jnp
from jax.experimental import pallas as pl
from jax.experimental.pallas import tpu as pltpu

_BM = 512


def _fused_kernel(mask_ref, x_ref, pt_ref, lh_ref, loss_ref, hm_ref,
                  acc_ref, hist_ref):
    i = pl.program_id(0)
    nsteps = pl.num_programs(0)
    x = x_ref[...]
    m = mask_ref[...]
    C = x.shape[1]

    rowmax = jnp.max(x, axis=1, keepdims=True)
    e = jnp.exp(x - rowmax)
    denom = jnp.sum(e, axis=1, keepdims=True)
    w = m / denom
    contrib = jnp.sum(e * w, axis=0, keepdims=True)

    iota = jax.lax.broadcasted_iota(jnp.int32, x.shape, 1)
    idx = jnp.min(jnp.where(x == rowmax, iota, C), axis=1, keepdims=True)
    onehot = (iota == idx).astype(x.dtype)
    hcontrib = jnp.sum(onehot * m, axis=0, keepdims=True)

    @pl.when(i == 0)
    def _():
        acc_ref[...] = jnp.zeros_like(acc_ref)
        hist_ref[...] = jnp.zeros_like(hist_ref)

    acc_ref[...] += contrib
    hist_ref[...] += hcontrib

    @pl.when(i == nsteps - 1)
    def _():
        hist = hist_ref[...]
        s = jnp.sum(hist)
        histogram = hist / s
        mean_probs = acc_ref[...] / s
        inv_lh = 1.0 / lh_ref[...]
        sc_pt = jnp.where(jnp.isinf(inv_lh), 0.0, inv_lh)
        mod_pt = pt_ref[...] * sc_pt
        mod_pt = mod_pt / jnp.sum(mod_pt)
        inv_h = 1.0 / histogram
        sc_ps = jnp.where(jnp.isinf(inv_h), 0.0, inv_h)
        mod_ps = mean_probs * sc_ps
        mod_ps = mod_ps / jnp.sum(mod_ps)
        loss = jnp.sum(mod_pt * jnp.log(mod_ps + 1e-9))
        loss_ref[...] = loss.reshape(1, 1)
        hm_ref[...] = jnp.mean(histogram).reshape(1, 1)


def kernel(mask, logits_ulb_s, p_t, label_hist):
    B, C = logits_ulb_s.shape
    bm = _BM
    grid = B // bm
    dt = logits_ulb_s.dtype
    mask_f = mask.astype(dt).reshape(B, 1)
    pt2 = p_t.reshape(1, C)
    lh2 = label_hist.reshape(1, C)
    loss, hm = pl.pallas_call(
        _fused_kernel,
        grid=(grid,),
        in_specs=[
            pl.BlockSpec((bm, 1), lambda i: (i, 0)),
            pl.BlockSpec((bm, C), lambda i: (i, 0)),
            pl.BlockSpec((1, C), lambda i: (0, 0)),
            pl.BlockSpec((1, C), lambda i: (0, 0)),
        ],
        out_specs=[
            pl.BlockSpec((1, 1), lambda i: (0, 0)),
            pl.BlockSpec((1, 1), lambda i: (0, 0)),
        ],
        out_shape=[
            jax.ShapeDtypeStruct((1, 1), dt),
            jax.ShapeDtypeStruct((1, 1), dt),
        ],
        scratch_shapes=[
            pltpu.VMEM((1, C), dt),
            pltpu.VMEM((1, C), dt),
        ],
    )(mask_f, logits_ulb_s, pt2, lh2)
    return loss[0, 0], hm[0, 0]
